# baseline (speedup 1.0000x reference)
import functools

import jax
import jax.numpy as jnp
from jax import lax
from jax.experimental import pallas as pl
from jax.experimental.pallas import tpu as pltpu
from jax.experimental.pallas import tpu_sc as plsc

F = 16
B = 16384
NC = 2
NS = 16
NW = NC * NS
BPW = B // NW
CH = 128
NCHUNK = BPW // CH


def _sc_body(urow_h, ucol_h, irow_h, icol_h, ug_h, ig_h, um_h, im_h,
             gmf_o, um_o, im_o,
             urow_v, ucol_v, irow_v, icol_v,
             ug_r, ig_r, um_r, im_r,
             gm_s, um_s, im_s, sem):
    wid = lax.axis_index("s") * NC + lax.axis_index("c")
    base = wid * BPW
    pltpu.sync_copy(urow_h.at[pl.ds(base, BPW)], urow_v)
    pltpu.sync_copy(ucol_h.at[pl.ds(base, BPW)], ucol_v)
    pltpu.sync_copy(irow_h.at[pl.ds(base, BPW)], irow_v)
    pltpu.sync_copy(icol_h.at[pl.ds(base, BPW)], icol_v)

    for c in range(NCHUNK):
        r0 = c * CH
        c0 = pltpu.async_copy(ug_h.at[urow_v.at[pl.ds(r0, CH)]], ug_r, sem)
        c1 = pltpu.async_copy(ig_h.at[irow_v.at[pl.ds(r0, CH)]], ig_r, sem)
        c2 = pltpu.async_copy(um_h.at[urow_v.at[pl.ds(r0, CH)]], um_r, sem)
        c3 = pltpu.async_copy(im_h.at[irow_v.at[pl.ds(r0, CH)]], im_r, sem)
        c0.wait()
        c1.wait()
        c2.wait()
        c3.wait()

        def extract(g, carry, r0=r0):
            row0 = g * 16
            offs_u = ucol_v[pl.ds(r0 + row0, 16)]
            offs_i = icol_v[pl.ds(r0 + row0, 16)]
            for k in range(16):
                j = row0 + k
                ou = offs_u[k]
                oi = offs_i[k]
                ug = ug_r[j, pl.ds(ou, F)]
                ig = ig_r[j, pl.ds(oi, F)]
                gm_s[j] = ug * ig
                um_s[j] = um_r[j, pl.ds(ou, F)]
                im_s[j] = im_r[j, pl.ds(oi, F)]
            return carry

        lax.fori_loop(0, CH // 16, extract, 0)
        pltpu.sync_copy(gm_s, gmf_o.at[pl.ds(base + r0, CH)])
        pltpu.sync_copy(um_s, um_o.at[pl.ds(base + r0, CH)])
        pltpu.sync_copy(im_s, im_o.at[pl.ds(base + r0, CH)])


_sc_gather = functools.partial(
    pl.kernel,
    mesh=plsc.VectorSubcoreMesh(core_axis_name="c", subcore_axis_name="s"),
    out_type=[
        jax.ShapeDtypeStruct((B, F), jnp.float32),
        jax.ShapeDtypeStruct((B, F), jnp.float32),
        jax.ShapeDtypeStruct((B, F), jnp.float32),
    ],
    scratch_types=[
        pltpu.VMEM((BPW,), jnp.int32),
        pltpu.VMEM((BPW,), jnp.int32),
        pltpu.VMEM((BPW,), jnp.int32),
        pltpu.VMEM((BPW,), jnp.int32),
        pltpu.VMEM((CH, 8 * F), jnp.float32),
        pltpu.VMEM((CH, 8 * F), jnp.float32),
        pltpu.VMEM((CH, 8 * F), jnp.float32),
        pltpu.VMEM((CH, 8 * F), jnp.float32),
        pltpu.VMEM((CH, F), jnp.float32),
        pltpu.VMEM((CH, F), jnp.float32),
        pltpu.VMEM((CH, F), jnp.float32),
        pltpu.SemaphoreType.DMA,
    ],
)(_sc_body)


BM = 2048


def _tc_body(gmf_ref, um_ref, im_ref, w1_ref, b1_ref, w2_ref, b2_ref,
             wog_ref, woh_ref, bo_ref, out_ref):
    mlp_in = jnp.concatenate([um_ref[...], im_ref[...]], axis=1)
    h = jnp.dot(mlp_in, w1_ref[...], preferred_element_type=jnp.float32)
    h = jnp.maximum(h + b1_ref[...], 0.0)
    h = jnp.dot(h, w2_ref[...], preferred_element_type=jnp.float32)
    h = jnp.maximum(h + b2_ref[...], 0.0)
    s = jnp.dot(gmf_ref[...], wog_ref[...], preferred_element_type=jnp.float32)
    s = s + jnp.dot(h, woh_ref[...], preferred_element_type=jnp.float32)
    out_ref[...] = s + bo_ref[...]


def _tc_mlp(gmf, um, im, W1, b1, W2, b2, Wo, bo):
    grid = (B // BM,)
    full = lambda shape: pl.BlockSpec(shape, lambda i: (0, 0))
    return pl.pallas_call(
        _tc_body,
        grid=grid,
        in_specs=[
            pl.BlockSpec((BM, F), lambda i: (i, 0)),
            pl.BlockSpec((BM, F), lambda i: (i, 0)),
            pl.BlockSpec((BM, F), lambda i: (i, 0)),
            full((2 * F, 2 * F)),
            full((1, 2 * F)),
            full((2 * F, F)),
            full((1, F)),
            full((F, 1)),
            full((F, 1)),
            full((1, 1)),
        ],
        out_specs=pl.BlockSpec((BM, 1), lambda i: (i, 0)),
        out_shape=jax.ShapeDtypeStruct((B, 1), jnp.float32),
    )(gmf, um, im, W1, b1.reshape(1, -1), W2, b2.reshape(1, -1),
      Wo[:F], Wo[F:], bo.reshape(1, 1))


def kernel(users, items, user_gmf, item_gmf, user_mlp, item_mlp,
           W1, b1, W2, b2, Wo, bo):
    users = users.astype(jnp.int32)
    items = items.astype(jnp.int32)
    urow, ucol = users >> 3, (users & 7) * F
    irow, icol = items >> 3, (items & 7) * F
    ug2 = user_gmf.reshape(-1, 8 * F)
    ig2 = item_gmf.reshape(-1, 8 * F)
    um2 = user_mlp.reshape(-1, 8 * F)
    im2 = item_mlp.reshape(-1, 8 * F)
    gmf, um, im = _sc_gather(urow, ucol, irow, icol, ug2, ig2, um2, im2)
    scores = _tc_mlp(gmf, um, im, W1, b1, W2, b2, Wo, bo)
    return scores[:, 0]
